# Initial kernel scaffold; baseline (speedup 1.0000x reference)
import jax
import jax.numpy as jnp
from jax.experimental import pallas as pl


def kernel(entity_emb, item_emb, kg_rel, kg_neigh, kg_dst, ii_src, ii_dst, weight):
    raise NotImplementedError("write your pallas kernel here")


# speedup vs baseline: 1.4339x; 1.4339x over previous
import functools

import jax
import jax.numpy as jnp
from jax import lax
from jax.experimental import pallas as pl
from jax.experimental.pallas import tpu as pltpu
from jax.experimental.pallas import tpu_sc as plsc

N_ENT = 50000
N_ITEM = 10000
D = 128
N_REL = 12
R = N_REL - 1
E_KG = 500000
E_II = 320000

NC = 2
NS = 16
NW = NC * NS

KG_CHUNK = 125
KG_GROUPS = 5
KG_GCHUNK = 25
II_CHUNK = 125
II_GROUPS = 4
II_GCHUNK = 20

CB = 4
CW = 32
ENT_PAD = 50048
ITEM_PAD = 10240
ENT_ROWS_PER_TILE = ENT_PAD // NS
ITEM_ROWS_PER_TILE = ITEM_PAD // NS
ENT_ZCHUNK = 184
ITEM_ZCHUNK = 64

_mesh = plsc.VectorSubcoreMesh(core_axis_name="c", subcore_axis_name="s")
_sc_params = pltpu.CompilerParams(use_tc_tiling_on_sc=False)


@functools.partial(
    pl.kernel,
    out_type=jax.ShapeDtypeStruct((NC, CB, ENT_PAD, CW), jnp.float32),
    mesh=_mesh,
    compiler_params=_sc_params,
    scratch_types=[
        pltpu.VMEM((KG_GCHUNK, KG_CHUNK), jnp.int32),
        pltpu.VMEM((KG_GCHUNK, KG_CHUNK), jnp.int32),
        pltpu.VMEM((KG_CHUNK, CW), jnp.float32),
        pltpu.VMEM((ENT_ZCHUNK, CW), jnp.float32),
        pltpu.VMEM_SHARED((ENT_PAD, CW), jnp.float32),
        pltpu.SemaphoreType.DMA,
    ],
)
def _kg_pass(t_hbm, comb_hbm, dst_hbm, out_hbm, comb_v, dst_v, vals, zbuf,
             acc, sem):
    c = lax.axis_index("c")
    s = lax.axis_index("s")
    wid = c * NS + s

    zero = jnp.zeros((16,), jnp.float32)

    @pl.loop(0, ENT_ZCHUNK)
    def _(i):
        zbuf[i, pl.ds(0, 16)] = zero
        zbuf[i, pl.ds(16, 16)] = zero

    for p in range(CB):
        @pl.loop(0, ENT_ROWS_PER_TILE // ENT_ZCHUNK)
        def _(j):
            pltpu.sync_copy(
                zbuf, acc.at[pl.ds(s * ENT_ROWS_PER_TILE + j * ENT_ZCHUNK,
                                   ENT_ZCHUNK)])
        plsc.subcore_barrier()

        @pl.loop(0, KG_GROUPS)
        def _(g):
            pltpu.sync_copy(comb_hbm.at[wid].at[g], comb_v)
            pltpu.sync_copy(dst_hbm.at[wid].at[g], dst_v)

            @pl.loop(0, KG_GCHUNK)
            def _(j):
                pltpu.async_copy(t_hbm.at[p].at[comb_v.at[j]], vals,
                                 sem).wait()
                pltpu.sync_copy(vals, acc.at[dst_v.at[j]], add=True)
        plsc.subcore_barrier()

        pltpu.sync_copy(
            acc.at[pl.ds(s * ENT_ROWS_PER_TILE, ENT_ROWS_PER_TILE)],
            out_hbm.at[c].at[p].at[pl.ds(s * ENT_ROWS_PER_TILE,
                                         ENT_ROWS_PER_TILE)])


@functools.partial(
    pl.kernel,
    out_type=jax.ShapeDtypeStruct((NC, ITEM_PAD, D), jnp.float32),
    mesh=_mesh,
    compiler_params=_sc_params,
    scratch_types=[
        pltpu.VMEM((II_GCHUNK, II_CHUNK), jnp.int32),
        pltpu.VMEM((II_GCHUNK, II_CHUNK), jnp.int32),
        pltpu.VMEM((II_CHUNK, D), jnp.float32),
        pltpu.VMEM((ITEM_ZCHUNK, D), jnp.float32),
        pltpu.VMEM_SHARED((ITEM_PAD, D), jnp.float32),
        pltpu.SemaphoreType.DMA,
    ],
)
def _ii_pass(emb_hbm, src_hbm, dst_hbm, out_hbm, src_v, dst_v, vals, zbuf,
             acc, sem):
    c = lax.axis_index("c")
    s = lax.axis_index("s")
    wid = c * NS + s

    zero = jnp.zeros((16,), jnp.float32)

    @pl.loop(0, ITEM_ZCHUNK)
    def _(i):
        for q in range(D // 16):
            zbuf[i, pl.ds(q * 16, 16)] = zero

    @pl.loop(0, ITEM_ROWS_PER_TILE // ITEM_ZCHUNK)
    def _(j):
        pltpu.sync_copy(
            zbuf, acc.at[pl.ds(s * ITEM_ROWS_PER_TILE + j * ITEM_ZCHUNK,
                               ITEM_ZCHUNK)])
    plsc.subcore_barrier()

    @pl.loop(0, II_GROUPS)
    def _(g):
        pltpu.sync_copy(src_hbm.at[wid].at[g], src_v)
        pltpu.sync_copy(dst_hbm.at[wid].at[g], dst_v)

        @pl.loop(0, II_GCHUNK)
        def _(j):
            pltpu.async_copy(emb_hbm.at[src_v.at[j]], vals, sem).wait()
            pltpu.sync_copy(vals, acc.at[dst_v.at[j]], add=True)
    plsc.subcore_barrier()

    pltpu.sync_copy(
        acc.at[pl.ds(s * ITEM_ROWS_PER_TILE, ITEM_ROWS_PER_TILE)],
        out_hbm.at[c].at[pl.ds(s * ITEM_ROWS_PER_TILE, ITEM_ROWS_PER_TILE)])


_TB_CHUNK = 2000


def _t_build_body(w_ref, e_ref, o_ref):
    r = pl.program_id(0)
    wrow = w_ref[pl.ds(r, 1), :]
    prod = e_ref[...] * wrow
    for p in range(CB):
        o_ref[p] = prod[:, p * CW:(p + 1) * CW]


def _build_t(weight, e):
    nsteps = N_ENT // _TB_CHUNK
    return pl.pallas_call(
        _t_build_body,
        grid=(R, nsteps),
        in_specs=[
            pl.BlockSpec((R, D), lambda r, n: (0, 0)),
            pl.BlockSpec((_TB_CHUNK, D), lambda r, n: (n, 0)),
        ],
        out_specs=pl.BlockSpec((CB, _TB_CHUNK, CW),
                               lambda r, n: (0, r * nsteps + n, 0)),
        out_shape=jax.ShapeDtypeStruct((CB, R * N_ENT, CW), jnp.float32),
    )(weight, e)


_EN_CHUNK = 2000


def _ent_norm_body(p_ref, o_ref):
    x = p_ref[0] + p_ref[1]
    cols = jnp.concatenate([x[p] for p in range(CB)], axis=-1)
    norm = jnp.sqrt(jnp.sum(cols * cols, axis=-1, keepdims=True))
    o_ref[...] = cols / jnp.maximum(norm, 1e-12)


def _ent_combine_norm(parts):
    nsteps = N_ENT // _EN_CHUNK
    return pl.pallas_call(
        _ent_norm_body,
        grid=(nsteps,),
        in_specs=[pl.BlockSpec((NC, CB, _EN_CHUNK, CW),
                               lambda n: (0, 0, n, 0))],
        out_specs=pl.BlockSpec((_EN_CHUNK, D), lambda n: (n, 0)),
        out_shape=jax.ShapeDtypeStruct((N_ENT, D), jnp.float32),
    )(parts)


_IN_CHUNK = 2000


def _item_norm_body(p_ref, o_ref):
    x = p_ref[0] + p_ref[1]
    norm = jnp.sqrt(jnp.sum(x * x, axis=-1, keepdims=True))
    o_ref[...] = x / jnp.maximum(norm, 1e-12)


def _item_combine_norm(parts):
    nsteps = N_ITEM // _IN_CHUNK
    return pl.pallas_call(
        _item_norm_body,
        grid=(nsteps,),
        in_specs=[pl.BlockSpec((NC, _IN_CHUNK, D), lambda n: (0, n, 0))],
        out_specs=pl.BlockSpec((_IN_CHUNK, D), lambda n: (n, 0)),
        out_shape=jax.ShapeDtypeStruct((N_ITEM, D), jnp.float32),
    )(parts)


def kernel(entity_emb, item_emb, kg_rel, kg_neigh, kg_dst, ii_src, ii_dst,
           weight):
    rel = kg_rel.astype(jnp.int32)
    neigh = kg_neigh.astype(jnp.int32)
    comb = (rel * N_ENT + neigh).reshape(NW, KG_GROUPS, KG_GCHUNK, KG_CHUNK)
    kgd = kg_dst.astype(jnp.int32).reshape(NW, KG_GROUPS, KG_GCHUNK, KG_CHUNK)
    iis = ii_src.astype(jnp.int32).reshape(NW, II_GROUPS, II_GCHUNK, II_CHUNK)
    iid = ii_dst.astype(jnp.int32).reshape(NW, II_GROUPS, II_GCHUNK, II_CHUNK)

    ii_parts = _ii_pass(item_emb, iis, iid)[:, :N_ITEM, :]
    ia = _item_combine_norm(ii_parts)

    e = entity_emb
    ent_out = [entity_emb]
    for _ in range(2):
        t = _build_t(weight, e)
        parts = _kg_pass(t, comb, kgd)[:, :, :N_ENT, :]
        e = _ent_combine_norm(parts)
        ent_out.append(e)

    return (jnp.stack(ent_out), jnp.stack([item_emb, ia, ia]))


# speedup vs baseline: 1.6662x; 1.1620x over previous
import functools

import jax
import jax.numpy as jnp
from jax import lax
from jax.experimental import pallas as pl
from jax.experimental.pallas import tpu as pltpu
from jax.experimental.pallas import tpu_sc as plsc

N_ENT = 50000
N_ITEM = 10000
D = 128
N_REL = 12
R = N_REL - 1
E_KG = 500000
E_II = 320000

NC = 2
NS = 16
NW = NC * NS

KG_CHUNK = 125
KG_GROUPS = 5
KG_GCHUNK = 25
KG_K = 5
II_CHUNK = 125
II_GROUPS = 4
II_GCHUNK = 20
II_K = 2

CB = 4
CW = 32
ENT_PAD = 50048
ITEM_PAD = 10240
ENT_ROWS_PER_TILE = ENT_PAD // NS
ITEM_ROWS_PER_TILE = ITEM_PAD // NS
ENT_ZCHUNK = 136
ITEM_ZCHUNK = 64

_mesh = plsc.VectorSubcoreMesh(core_axis_name="c", subcore_axis_name="s")
_sc_params = pltpu.CompilerParams(use_tc_tiling_on_sc=False)


@functools.partial(
    pl.kernel,
    out_type=jax.ShapeDtypeStruct((NC, CB, ENT_PAD, CW), jnp.float32),
    mesh=_mesh,
    compiler_params=_sc_params,
    scratch_types=[
        pltpu.VMEM((KG_GCHUNK, KG_CHUNK), jnp.int32),
        pltpu.VMEM((KG_GCHUNK, KG_CHUNK), jnp.int32),
        pltpu.VMEM((KG_K, KG_CHUNK, CW), jnp.float32),
        pltpu.VMEM((ENT_ZCHUNK, CW), jnp.float32),
        pltpu.VMEM_SHARED((ENT_PAD, CW), jnp.float32),
        pltpu.SemaphoreType.DMA,
        pltpu.SemaphoreType.DMA,
    ],
)
def _kg_pass(t_hbm, comb_hbm, dst_hbm, out_hbm, comb_v, dst_v, vals, zbuf,
             acc, gsem, ssem):
    c = lax.axis_index("c")
    s = lax.axis_index("s")
    wid = c * NS + s

    zero = jnp.zeros((16,), jnp.float32)

    @pl.loop(0, ENT_ZCHUNK)
    def _(i):
        zbuf[i, pl.ds(0, 16)] = zero
        zbuf[i, pl.ds(16, 16)] = zero

    for p in range(CB):
        @pl.loop(0, ENT_ROWS_PER_TILE // ENT_ZCHUNK)
        def _(j):
            pltpu.sync_copy(
                zbuf, acc.at[pl.ds(s * ENT_ROWS_PER_TILE + j * ENT_ZCHUNK,
                                   ENT_ZCHUNK)])
        plsc.subcore_barrier()

        @pl.loop(0, KG_GROUPS)
        def _(g):
            pltpu.sync_copy(comb_hbm.at[wid].at[g], comb_v)
            pltpu.sync_copy(dst_hbm.at[wid].at[g], dst_v)

            @pl.loop(0, KG_GCHUNK // KG_K)
            def _(bb):
                gathers = []
                for b in range(KG_K):
                    gathers.append(pltpu.async_copy(
                        t_hbm.at[p].at[comb_v.at[bb * KG_K + b]],
                        vals.at[b], gsem))
                scatters = []
                for b in range(KG_K):
                    gathers[b].wait()
                    scatters.append(pltpu.async_copy(
                        vals.at[b], acc.at[dst_v.at[bb * KG_K + b]], ssem,
                        add=True))
                for b in range(KG_K):
                    scatters[b].wait()
        plsc.subcore_barrier()

        pltpu.sync_copy(
            acc.at[pl.ds(s * ENT_ROWS_PER_TILE, ENT_ROWS_PER_TILE)],
            out_hbm.at[c].at[p].at[pl.ds(s * ENT_ROWS_PER_TILE,
                                         ENT_ROWS_PER_TILE)])


@functools.partial(
    pl.kernel,
    out_type=jax.ShapeDtypeStruct((NC, ITEM_PAD, D), jnp.float32),
    mesh=_mesh,
    compiler_params=_sc_params,
    scratch_types=[
        pltpu.VMEM((II_GCHUNK, II_CHUNK), jnp.int32),
        pltpu.VMEM((II_GCHUNK, II_CHUNK), jnp.int32),
        pltpu.VMEM((II_K, II_CHUNK, D), jnp.float32),
        pltpu.VMEM((ITEM_ZCHUNK, D), jnp.float32),
        pltpu.VMEM_SHARED((ITEM_PAD, D), jnp.float32),
        pltpu.SemaphoreType.DMA,
        pltpu.SemaphoreType.DMA,
    ],
)
def _ii_pass(emb_hbm, src_hbm, dst_hbm, out_hbm, src_v, dst_v, vals, zbuf,
             acc, gsem, ssem):
    c = lax.axis_index("c")
    s = lax.axis_index("s")
    wid = c * NS + s

    zero = jnp.zeros((16,), jnp.float32)

    @pl.loop(0, ITEM_ZCHUNK)
    def _(i):
        for q in range(D // 16):
            zbuf[i, pl.ds(q * 16, 16)] = zero

    @pl.loop(0, ITEM_ROWS_PER_TILE // ITEM_ZCHUNK)
    def _(j):
        pltpu.sync_copy(
            zbuf, acc.at[pl.ds(s * ITEM_ROWS_PER_TILE + j * ITEM_ZCHUNK,
                               ITEM_ZCHUNK)])
    plsc.subcore_barrier()

    @pl.loop(0, II_GROUPS)
    def _(g):
        pltpu.sync_copy(src_hbm.at[wid].at[g], src_v)
        pltpu.sync_copy(dst_hbm.at[wid].at[g], dst_v)

        @pl.loop(0, II_GCHUNK // II_K)
        def _(bb):
            gathers = []
            for b in range(II_K):
                gathers.append(pltpu.async_copy(
                    emb_hbm.at[src_v.at[bb * II_K + b]], vals.at[b], gsem))
            scatters = []
            for b in range(II_K):
                gathers[b].wait()
                scatters.append(pltpu.async_copy(
                    vals.at[b], acc.at[dst_v.at[bb * II_K + b]], ssem,
                    add=True))
            for b in range(II_K):
                scatters[b].wait()
    plsc.subcore_barrier()

    pltpu.sync_copy(
        acc.at[pl.ds(s * ITEM_ROWS_PER_TILE, ITEM_ROWS_PER_TILE)],
        out_hbm.at[c].at[pl.ds(s * ITEM_ROWS_PER_TILE, ITEM_ROWS_PER_TILE)])


_TB_CHUNK = 2000


def _t_build_body(w_ref, e_ref, o_ref):
    r = pl.program_id(0)
    wrow = w_ref[pl.ds(r, 1), :]
    prod = e_ref[...] * wrow
    for p in range(CB):
        o_ref[p] = prod[:, p * CW:(p + 1) * CW]


def _build_t(weight, e):
    nsteps = N_ENT // _TB_CHUNK
    return pl.pallas_call(
        _t_build_body,
        grid=(R, nsteps),
        in_specs=[
            pl.BlockSpec((R, D), lambda r, n: (0, 0)),
            pl.BlockSpec((_TB_CHUNK, D), lambda r, n: (n, 0)),
        ],
        out_specs=pl.BlockSpec((CB, _TB_CHUNK, CW),
                               lambda r, n: (0, r * nsteps + n, 0)),
        out_shape=jax.ShapeDtypeStruct((CB, R * N_ENT, CW), jnp.float32),
    )(weight, e)


_EN_CHUNK = 2000


def _ent_norm_body(p_ref, o_ref):
    x = p_ref[0] + p_ref[1]
    cols = jnp.concatenate([x[p] for p in range(CB)], axis=-1)
    norm = jnp.sqrt(jnp.sum(cols * cols, axis=-1, keepdims=True))
    o_ref[...] = cols / jnp.maximum(norm, 1e-12)


def _ent_combine_norm(parts):
    nsteps = N_ENT // _EN_CHUNK
    return pl.pallas_call(
        _ent_norm_body,
        grid=(nsteps,),
        in_specs=[pl.BlockSpec((NC, CB, _EN_CHUNK, CW),
                               lambda n: (0, 0, n, 0))],
        out_specs=pl.BlockSpec((_EN_CHUNK, D), lambda n: (n, 0)),
        out_shape=jax.ShapeDtypeStruct((N_ENT, D), jnp.float32),
    )(parts)


_IN_CHUNK = 2000


def _item_norm_body(p_ref, o_ref):
    x = p_ref[0] + p_ref[1]
    norm = jnp.sqrt(jnp.sum(x * x, axis=-1, keepdims=True))
    o_ref[...] = x / jnp.maximum(norm, 1e-12)


def _item_combine_norm(parts):
    nsteps = N_ITEM // _IN_CHUNK
    return pl.pallas_call(
        _item_norm_body,
        grid=(nsteps,),
        in_specs=[pl.BlockSpec((NC, _IN_CHUNK, D), lambda n: (0, n, 0))],
        out_specs=pl.BlockSpec((_IN_CHUNK, D), lambda n: (n, 0)),
        out_shape=jax.ShapeDtypeStruct((N_ITEM, D), jnp.float32),
    )(parts)


def kernel(entity_emb, item_emb, kg_rel, kg_neigh, kg_dst, ii_src, ii_dst,
           weight):
    rel = kg_rel.astype(jnp.int32)
    neigh = kg_neigh.astype(jnp.int32)
    comb = (rel * N_ENT + neigh).reshape(NW, KG_GROUPS, KG_GCHUNK, KG_CHUNK)
    kgd = kg_dst.astype(jnp.int32).reshape(NW, KG_GROUPS, KG_GCHUNK, KG_CHUNK)
    iis = ii_src.astype(jnp.int32).reshape(NW, II_GROUPS, II_GCHUNK, II_CHUNK)
    iid = ii_dst.astype(jnp.int32).reshape(NW, II_GROUPS, II_GCHUNK, II_CHUNK)

    ii_parts = _ii_pass(item_emb, iis, iid)[:, :N_ITEM, :]
    ia = _item_combine_norm(ii_parts)

    e = entity_emb
    ent_out = [entity_emb]
    for _ in range(2):
        t = _build_t(weight, e)
        parts = _kg_pass(t, comb, kgd)[:, :, :N_ENT, :]
        e = _ent_combine_norm(parts)
        ent_out.append(e)

    return (jnp.stack(ent_out), jnp.stack([item_emb, ia, ia]))
